# Initial kernel scaffold; baseline (speedup 1.0000x reference)
import jax
import jax.numpy as jnp
from jax.experimental import pallas as pl


def kernel(features, point_idx, vecter):
    raise NotImplementedError("write your pallas kernel here")


# speedup vs baseline: 2.6449x; 2.6449x over previous
import functools

import jax
import jax.numpy as jnp
from jax.experimental import pallas as pl
from jax.experimental.pallas import tpu as pltpu

_N = 200000
_D = 256
_B = 32
_BR = 2000


def _body(idx_ref, feat_ref, v_ref, out_ref):
    v = v_ref[...]
    v = v - jnp.max(v, axis=1, keepdims=True)
    e = jnp.exp(v)
    v_sm = e / jnp.sum(e, axis=1, keepdims=True)

    idx = idx_ref[...]
    cols = jax.lax.broadcasted_iota(jnp.int32, (_BR, _B), 1)
    onehot = jnp.where(idx == cols, 1.0, 0.0).astype(jnp.float32)
    gathered = jnp.dot(onehot, v_sm, preferred_element_type=jnp.float32)
    out_ref[...] = feat_ref[...] * gathered


def kernel(features, point_idx, vecter):
    idx2d = point_idx.astype(jnp.int32).reshape(_N, 1)
    grid = _N // _BR
    return pl.pallas_call(
        _body,
        grid=(grid,),
        in_specs=[
            pl.BlockSpec((_BR, 1), lambda i: (i, 0)),
            pl.BlockSpec((_BR, _D), lambda i: (i, 0)),
            pl.BlockSpec((_B, _D), lambda i: (0, 0)),
        ],
        out_specs=pl.BlockSpec((_BR, _D), lambda i: (i, 0)),
        out_shape=jax.ShapeDtypeStruct((_N, _D), jnp.float32),
    )(idx2d, features, vecter)


# speedup vs baseline: 2.8704x; 1.0853x over previous
import functools

import jax
import jax.numpy as jnp
from jax.experimental import pallas as pl
from jax.experimental.pallas import tpu as pltpu

_N = 200000
_D = 256
_B = 32
_BR = 8000


def _body(idx_ref, feat_ref, v_ref, out_ref):
    v = v_ref[...]
    v = v - jnp.max(v, axis=1, keepdims=True)
    e = jnp.exp(v)
    v_sm = e / jnp.sum(e, axis=1, keepdims=True)

    idx = idx_ref[...]
    cols = jax.lax.broadcasted_iota(jnp.int32, (_BR, _B), 1)
    onehot = jnp.where(idx == cols, 1.0, 0.0).astype(jnp.float32)
    gathered = jnp.dot(onehot, v_sm, preferred_element_type=jnp.float32)
    out_ref[...] = feat_ref[...] * gathered


def kernel(features, point_idx, vecter):
    idx2d = point_idx.astype(jnp.int32).reshape(_N, 1)
    grid = _N // _BR
    return pl.pallas_call(
        _body,
        grid=(grid,),
        in_specs=[
            pl.BlockSpec((_BR, 1), lambda i: (i, 0)),
            pl.BlockSpec((_BR, _D), lambda i: (i, 0)),
            pl.BlockSpec((_B, _D), lambda i: (0, 0)),
        ],
        out_specs=pl.BlockSpec((_BR, _D), lambda i: (i, 0)),
        out_shape=jax.ShapeDtypeStruct((_N, _D), jnp.float32),
    )(idx2d, features, vecter)


# speedup vs baseline: 5.0437x; 1.7572x over previous
import functools

import jax
import jax.numpy as jnp
from jax.experimental import pallas as pl
from jax.experimental.pallas import tpu as pltpu

_N = 200000
_D = 256
_B = 32
_BR = 8000


def _body(idx_ref, feat_ref, v_ref, out_ref):
    v = v_ref[...]
    v = v - jnp.max(v, axis=1, keepdims=True)
    e = jnp.exp(v)
    v_sm = e / jnp.sum(e, axis=1, keepdims=True)

    idx = idx_ref[0]
    rows = jax.lax.broadcasted_iota(jnp.int32, (_B, _BR), 0)
    onehot_t = jnp.where(idx == rows, 1.0, 0.0).astype(jnp.float32)
    gathered = jax.lax.dot_general(
        onehot_t, v_sm, (((0,), (0,)), ((), ())),
        preferred_element_type=jnp.float32)
    out_ref[...] = feat_ref[...] * gathered


def kernel(features, point_idx, vecter):
    grid = _N // _BR
    idx3d = point_idx.astype(jnp.int32).reshape(grid, 1, _BR)
    return pl.pallas_call(
        _body,
        grid=(grid,),
        in_specs=[
            pl.BlockSpec((1, 1, _BR), lambda i: (i, 0, 0)),
            pl.BlockSpec((_BR, _D), lambda i: (i, 0)),
            pl.BlockSpec((_B, _D), lambda i: (0, 0)),
        ],
        out_specs=pl.BlockSpec((_BR, _D), lambda i: (i, 0)),
        out_shape=jax.ShapeDtypeStruct((_N, _D), jnp.float32),
    )(idx3d, features, vecter)


# speedup vs baseline: 5.0604x; 1.0033x over previous
import functools

import jax
import jax.numpy as jnp
from jax.experimental import pallas as pl
from jax.experimental.pallas import tpu as pltpu

_N = 200000
_D = 256
_B = 32
_BR = 10000


def _body(idx_ref, feat_ref, v_ref, out_ref):
    v = v_ref[...]
    v = v - jnp.max(v, axis=1, keepdims=True)
    e = jnp.exp(v)
    v_sm = e / jnp.sum(e, axis=1, keepdims=True)

    idx = idx_ref[0]
    rows = jax.lax.broadcasted_iota(jnp.int32, (_B, _BR), 0)
    onehot_t = jnp.where(idx == rows, 1.0, 0.0).astype(jnp.float32)
    gathered = jax.lax.dot_general(
        onehot_t, v_sm, (((0,), (0,)), ((), ())),
        preferred_element_type=jnp.float32)
    out_ref[...] = feat_ref[...] * gathered


def kernel(features, point_idx, vecter):
    grid = _N // _BR
    idx3d = point_idx.astype(jnp.int32).reshape(grid, 1, _BR)
    return pl.pallas_call(
        _body,
        grid=(grid,),
        in_specs=[
            pl.BlockSpec((1, 1, _BR), lambda i: (i, 0, 0)),
            pl.BlockSpec((_BR, _D), lambda i: (i, 0)),
            pl.BlockSpec((_B, _D), lambda i: (0, 0)),
        ],
        out_specs=pl.BlockSpec((_BR, _D), lambda i: (i, 0)),
        out_shape=jax.ShapeDtypeStruct((_N, _D), jnp.float32),
    )(idx3d, features, vecter)
